# baseline (speedup 1.0000x reference)
import functools

import jax
import jax.numpy as jnp
from jax import lax
from jax.experimental import pallas as pl
from jax.experimental.pallas import tpu as pltpu
from jax.experimental.pallas import tpu_sc as plsc

N_EXP = 64
TOPK = 8
LANES = 16


@functools.partial(
    pl.kernel,
    out_type=jax.ShapeDtypeStruct((N_EXP,), jnp.float32),
    mesh=plsc.VectorSubcoreMesh(core_axis_name="c", subcore_axis_name="s"),
    scratch_types=[
        pltpu.VMEM((LANES,), jnp.int32),
        pltpu.VMEM((N_EXP,), jnp.float32),
    ],
)
def _gate_sc(attrs_hbm, out_hbm, attrs_v, out_v):
    cid = lax.axis_index("c")
    sid = lax.axis_index("s")

    @pl.when(jnp.logical_and(cid == 0, sid == 0))
    def _():
        pltpu.sync_copy(attrs_hbm, attrs_v)
        lane = lax.iota(jnp.int32, LANES)
        av = attrs_v[...]
        combined = lax.rem(av[0] * TOPK + av[1], N_EXP)
        eighth = jnp.full((LANES,), 1.0 / TOPK, dtype=jnp.float32)
        zero = jnp.zeros((LANES,), dtype=jnp.float32)
        for j in range(N_EXP // LANES):
            dist = lax.rem(lane + (j * LANES + N_EXP) - combined, N_EXP)
            out_v[pl.ds(j * LANES, LANES)] = jnp.where(dist < TOPK, eighth, zero)
        pltpu.sync_copy(out_v, out_hbm)


def kernel(x, attrs):
    attrs16 = jnp.zeros((LANES,), jnp.int32).at[:2].set(attrs.astype(jnp.int32))
    combine_weights = _gate_sc(attrs16)
    return combine_weights, jnp.zeros((), jnp.float32)


# speedup vs baseline: 1.0689x; 1.0689x over previous
import functools

import jax
import jax.numpy as jnp
from jax import lax
from jax.experimental import pallas as pl
from jax.experimental.pallas import tpu as pltpu
from jax.experimental.pallas import tpu_sc as plsc

N_EXP = 64
TOPK = 8
LANES = 16


@functools.partial(
    pl.kernel,
    out_type=jax.ShapeDtypeStruct((N_EXP,), jnp.float32),
    mesh=plsc.VectorSubcoreMesh(
        core_axis_name="c", subcore_axis_name="s", num_cores=1
    ),
    scratch_types=[
        pltpu.VMEM((LANES,), jnp.int32),
        pltpu.VMEM((N_EXP,), jnp.float32),
    ],
)
def _gate_sc(attrs_hbm, out_hbm, attrs_v, out_v):
    sid = lax.axis_index("s")

    @pl.when(sid == 0)
    def _():
        pltpu.sync_copy(attrs_hbm, attrs_v.at[pl.ds(0, 2)])
        lane = lax.iota(jnp.int32, LANES)
        av = attrs_v[...]
        combined = lax.rem(av[0] * TOPK + av[1], N_EXP)
        eighth = jnp.full((LANES,), 1.0 / TOPK, dtype=jnp.float32)
        zero = jnp.zeros((LANES,), dtype=jnp.float32)
        for j in range(N_EXP // LANES):
            dist = lax.rem(lane + (j * LANES + N_EXP) - combined, N_EXP)
            out_v[pl.ds(j * LANES, LANES)] = jnp.where(dist < TOPK, eighth, zero)
        pltpu.sync_copy(out_v, out_hbm)


def kernel(x, attrs):
    combine_weights = _gate_sc(attrs.astype(jnp.int32))
    return combine_weights, jnp.zeros((), jnp.float32)


# speedup vs baseline: 1.0802x; 1.0106x over previous
import functools

import jax
import jax.numpy as jnp
from jax import lax
from jax.experimental import pallas as pl
from jax.experimental.pallas import tpu as pltpu
from jax.experimental.pallas import tpu_sc as plsc

N_EXP = 64
TOPK = 8
LANES = 16


@functools.partial(
    pl.kernel,
    out_type=jax.ShapeDtypeStruct((N_EXP,), jnp.float32),
    mesh=plsc.VectorSubcoreMesh(
        core_axis_name="c", subcore_axis_name="s", num_cores=1, num_subcores=1
    ),
    scratch_types=[
        pltpu.VMEM((LANES,), jnp.int32),
        pltpu.VMEM((N_EXP,), jnp.float32),
    ],
)
def _gate_sc(attrs_hbm, out_hbm, attrs_v, out_v):
    sid = lax.axis_index("s")

    @pl.when(sid == 0)
    def _():
        pltpu.sync_copy(attrs_hbm, attrs_v.at[pl.ds(0, 2)])
        lane = lax.iota(jnp.int32, LANES)
        av = attrs_v[...]
        combined = lax.rem(av[0] * TOPK + av[1], N_EXP)
        eighth = jnp.full((LANES,), 1.0 / TOPK, dtype=jnp.float32)
        zero = jnp.zeros((LANES,), dtype=jnp.float32)
        for j in range(N_EXP // LANES):
            dist = lax.rem(lane + (j * LANES + N_EXP) - combined, N_EXP)
            out_v[pl.ds(j * LANES, LANES)] = jnp.where(dist < TOPK, eighth, zero)
        pltpu.sync_copy(out_v, out_hbm)


def kernel(x, attrs):
    combine_weights = _gate_sc(attrs.astype(jnp.int32))
    return combine_weights, jnp.zeros((), jnp.float32)


# speedup vs baseline: 1.1431x; 1.0583x over previous
import functools

import jax
import jax.numpy as jnp
from jax import lax
from jax.experimental import pallas as pl
from jax.experimental.pallas import tpu as pltpu
from jax.experimental.pallas import tpu_sc as plsc

N_EXP = 64
TOPK = 8
LANES = 16


@functools.partial(
    pl.kernel,
    out_type=jax.ShapeDtypeStruct((N_EXP,), jnp.float32),
    mesh=plsc.ScalarSubcoreMesh(axis_name="c", num_cores=1),
    scratch_types=[
        pltpu.SMEM((2,), jnp.int32),
        pltpu.SMEM((N_EXP,), jnp.float32),
    ],
)
def _gate_sc(attrs_hbm, out_hbm, attrs_s, out_s):
    pltpu.sync_copy(attrs_hbm, attrs_s)
    combined = lax.rem(attrs_s[0] * TOPK + attrs_s[1], N_EXP)

    def body(i, carry):
        dist = lax.rem(i + N_EXP - combined, N_EXP)
        out_s[i] = jnp.where(dist < TOPK, 1.0 / TOPK, 0.0)
        return carry

    lax.fori_loop(0, N_EXP, body, 0)
    pltpu.sync_copy(out_s, out_hbm)


def kernel(x, attrs):
    combine_weights = _gate_sc(attrs.astype(jnp.int32))
    return combine_weights, jnp.zeros((), jnp.float32)


# speedup vs baseline: 1.1446x; 1.0013x over previous
import functools

import jax
import jax.numpy as jnp
from jax import lax
from jax.experimental import pallas as pl
from jax.experimental.pallas import tpu as pltpu
from jax.experimental.pallas import tpu_sc as plsc

N_EXP = 64
TOPK = 8
LANES = 16


@functools.partial(
    pl.kernel,
    out_type=jax.ShapeDtypeStruct((N_EXP,), jnp.float32),
    mesh=plsc.ScalarSubcoreMesh(axis_name="c", num_cores=1),
    scratch_types=[
        pltpu.SMEM((2,), jnp.int32),
        pltpu.SMEM((N_EXP,), jnp.float32),
        pltpu.SemaphoreType.DMA,
    ],
)
def _gate_sc(attrs_hbm, out_hbm, attrs_s, out_s, sem):
    cp = pltpu.make_async_copy(attrs_hbm, attrs_s, sem)
    cp.start()
    for i in range(N_EXP):
        out_s[i] = 0.0
    cp.wait()
    combined = lax.rem(attrs_s[0] * TOPK + attrs_s[1], N_EXP)
    for i in range(TOPK):
        out_s[lax.rem(combined + i, N_EXP)] = 1.0 / TOPK
    pltpu.sync_copy(out_s, out_hbm)


def kernel(x, attrs):
    combine_weights = _gate_sc(attrs.astype(jnp.int32))
    return combine_weights, jnp.zeros((), jnp.float32)


# speedup vs baseline: 1.1448x; 1.0002x over previous
import functools

import jax
import jax.numpy as jnp
from jax import lax
from jax.experimental import pallas as pl
from jax.experimental.pallas import tpu as pltpu
from jax.experimental.pallas import tpu_sc as plsc

N_EXP = 64
TOPK = 8
ATTR1_MULT = 8


@functools.partial(
    pl.kernel,
    out_type=jax.ShapeDtypeStruct((N_EXP,), jnp.float32),
    mesh=plsc.ScalarSubcoreMesh(axis_name="c", num_cores=1),
    scratch_types=[
        pltpu.SMEM((2,), jnp.int32),
        pltpu.SMEM((N_EXP,), jnp.float32),
        pltpu.SemaphoreType.DMA,
    ],
)
def _gate_sc(attrs_hbm, out_hbm, attrs_s, out_s, sem):
    cp = pltpu.make_async_copy(attrs_hbm, attrs_s, sem)
    cp.start()
    for i in range(N_EXP):
        out_s[i] = 0.0
    cp.wait()
    combined = lax.rem(attrs_s[0] * ATTR1_MULT + attrs_s[1], N_EXP)
    for i in range(TOPK):
        out_s[lax.rem(combined + i, N_EXP)] = 1.0 / TOPK
    pltpu.sync_copy(out_s, out_hbm)


def kernel(x, attrs):
    combine_weights = _gate_sc(attrs.astype(jnp.int32))
    return combine_weights, jnp.zeros((), jnp.float32)
